# baseline (speedup 1.0000x reference)
import functools

import jax
import jax.numpy as jnp
from jax import lax
from jax.experimental import pallas as pl
from jax.experimental.pallas import tpu as pltpu
from jax.experimental.pallas import tpu_sc as plsc

K = 32
P = 4
L = P * K
NC = 2
NS = 16
NW = NC * NS
CH = 128
TC_BLK = 2048


@functools.lru_cache(maxsize=None)
def _build_gather(B):
    BPW = B // NW

    mesh = plsc.VectorSubcoreMesh(core_axis_name="c", subcore_axis_name="s")

    @functools.partial(
        pl.kernel,
        mesh=mesh,
        compiler_params=pltpu.CompilerParams(use_tc_tiling_on_sc=True),
        out_type=(
            jax.ShapeDtypeStruct((B, L), jnp.float32),
            jax.ShapeDtypeStruct((B, L), jnp.float32),
        ),
        scratch_types=[
            pltpu.VMEM((CH,), jnp.int32),
            pltpu.VMEM((CH,), jnp.int32),
            pltpu.VMEM((CH, L), jnp.float32),
            pltpu.VMEM((CH, L), jnp.float32),
            pltpu.SemaphoreType.DMA,
            pltpu.SemaphoreType.DMA,
        ],
    )
    def sc_gather(uq_hbm, iq_hbm, ut_hbm, it_hbm, ubuf_hbm, ibuf_hbm,
                  uv, iv, uslab, islab, sem_u, sem_i):
        wid = lax.axis_index("s") * NC + lax.axis_index("c")
        base = wid * BPW
        for c in range(BPW // CH):
            off = base + c * CH
            pltpu.sync_copy(uq_hbm.at[pl.ds(off, CH)], uv)
            pltpu.sync_copy(iq_hbm.at[pl.ds(off, CH)], iv)
            cu = pltpu.async_copy(ut_hbm.at[uv], uslab, sem_u)
            ci = pltpu.async_copy(it_hbm.at[iv], islab, sem_i)
            cu.wait()
            pltpu.sync_copy(uslab, ubuf_hbm.at[pl.ds(off, CH)])
            ci.wait()
            pltpu.sync_copy(islab, ibuf_hbm.at[pl.ds(off, CH)])

    return sc_gather


def _tc_dense(u_ref, i_ref, ur_ref, ir_ref, wu_ref, wi_ref, b_ref, o_ref):
    su = jnp.dot(u_ref[...], wu_ref[...],
                 preferred_element_type=jnp.float32)
    si = jnp.dot(i_ref[...], wi_ref[...],
                 preferred_element_type=jnp.float32)
    lane = lax.broadcasted_iota(jnp.int32, (TC_BLK, P), 1)
    s = (jnp.sum(jnp.where(lane == ur_ref[...], su, 0.0), axis=1,
                 keepdims=True)
         + jnp.sum(jnp.where(lane == ir_ref[...], si, 0.0), axis=1,
                   keepdims=True))
    o_ref[...] = jnp.maximum(s + b_ref[0, 0], 0.0)


@functools.lru_cache(maxsize=None)
def _build_dense(B):
    nblk = B // TC_BLK
    return pl.pallas_call(
        _tc_dense,
        grid=(nblk,),
        in_specs=[
            pl.BlockSpec((TC_BLK, L), lambda i: (i, 0)),
            pl.BlockSpec((TC_BLK, L), lambda i: (i, 0)),
            pl.BlockSpec((TC_BLK, 1), lambda i: (i, 0)),
            pl.BlockSpec((TC_BLK, 1), lambda i: (i, 0)),
            pl.BlockSpec((L, P), lambda i: (0, 0)),
            pl.BlockSpec((L, P), lambda i: (0, 0)),
            pl.BlockSpec((1, 1), lambda i: (0, 0)),
        ],
        out_specs=pl.BlockSpec((TC_BLK, 1), lambda i: (i, 0)),
        out_shape=jax.ShapeDtypeStruct((B, 1), jnp.float32),
    )


def kernel(user_ids, item_ids, user_table, item_table, dense_w, dense_b):
    B = user_ids.shape[0]
    uids = user_ids.astype(jnp.int32).reshape(B)
    iids = item_ids.astype(jnp.int32).reshape(B)
    uq = uids // P
    iq = iids // P
    ur = (uids % P).reshape(B, 1)
    ir = (iids % P).reshape(B, 1)
    ut2 = user_table.reshape(user_table.shape[0] // P, L)
    it2 = item_table.reshape(item_table.shape[0] // P, L)
    eye = jnp.eye(P, dtype=jnp.float32)
    wu4 = jnp.kron(eye, dense_w[0:K, :])
    wi4 = jnp.kron(eye, dense_w[K:2 * K, :])
    ubuf, ibuf = _build_gather(B)(uq, iq, ut2, it2)
    return _build_dense(B)(ubuf, ibuf, ur, ir, wu4, wi4,
                           dense_b.reshape(1, 1))


# speedup vs baseline: 1.0437x; 1.0437x over previous
import functools

import jax
import jax.numpy as jnp
from jax import lax
from jax.experimental import pallas as pl
from jax.experimental.pallas import tpu as pltpu
from jax.experimental.pallas import tpu_sc as plsc

K = 32
NC = 2
NS = 16
NW = NC * NS
TC_BLK = 2048


@functools.lru_cache(maxsize=None)
def _build_gather(B):
    BPW = B // NW

    mesh = plsc.VectorSubcoreMesh(core_axis_name="c", subcore_axis_name="s")

    @functools.partial(
        pl.kernel,
        mesh=mesh,
        compiler_params=pltpu.CompilerParams(use_tc_tiling_on_sc=False),
        out_type=(
            jax.ShapeDtypeStruct((B, K), jnp.float32),
            jax.ShapeDtypeStruct((B, K), jnp.float32),
        ),
        scratch_types=[
            pltpu.VMEM((BPW,), jnp.int32),
            pltpu.VMEM((BPW,), jnp.int32),
            pltpu.VMEM((BPW, K), jnp.float32),
            pltpu.VMEM((BPW, K), jnp.float32),
            pltpu.SemaphoreType.DMA,
            pltpu.SemaphoreType.DMA,
        ],
    )
    def sc_gather(uids_hbm, iids_hbm, ut_hbm, it_hbm, ubuf_hbm, ibuf_hbm,
                  uv, iv, urows, irows, sem_u, sem_i):
        wid = lax.axis_index("s") * NC + lax.axis_index("c")
        base = wid * BPW
        pltpu.sync_copy(uids_hbm.at[pl.ds(base, BPW)], uv)
        pltpu.sync_copy(iids_hbm.at[pl.ds(base, BPW)], iv)
        cu = pltpu.async_copy(ut_hbm.at[uv], urows, sem_u)
        ci = pltpu.async_copy(it_hbm.at[iv], irows, sem_i)
        cu.wait()
        pltpu.sync_copy(urows, ubuf_hbm.at[pl.ds(base, BPW)])
        ci.wait()
        pltpu.sync_copy(irows, ibuf_hbm.at[pl.ds(base, BPW)])

    return sc_gather


def _tc_dense(u_ref, i_ref, w_ref, b_ref, o_ref):
    wu = w_ref[0:K, :]
    wi = w_ref[K:2 * K, :]
    s = jnp.dot(u_ref[...], wu, preferred_element_type=jnp.float32)
    s = s + jnp.dot(i_ref[...], wi, preferred_element_type=jnp.float32)
    o_ref[...] = jnp.maximum(s + b_ref[0, 0], 0.0)


@functools.lru_cache(maxsize=None)
def _build_dense(B):
    nblk = B // TC_BLK
    return pl.pallas_call(
        _tc_dense,
        grid=(nblk,),
        in_specs=[
            pl.BlockSpec((TC_BLK, K), lambda i: (i, 0)),
            pl.BlockSpec((TC_BLK, K), lambda i: (i, 0)),
            pl.BlockSpec((2 * K, 1), lambda i: (0, 0)),
            pl.BlockSpec((1, 1), lambda i: (0, 0)),
        ],
        out_specs=pl.BlockSpec((TC_BLK, 1), lambda i: (i, 0)),
        out_shape=jax.ShapeDtypeStruct((B, 1), jnp.float32),
    )


def kernel(user_ids, item_ids, user_table, item_table, dense_w, dense_b):
    B = user_ids.shape[0]
    uids = user_ids.astype(jnp.int32).reshape(B)
    iids = item_ids.astype(jnp.int32).reshape(B)
    ubuf, ibuf = _build_gather(B)(uids, iids, user_table, item_table)
    return _build_dense(B)(ubuf, ibuf, dense_w, dense_b.reshape(1, 1))
